# baseline (speedup 1.0000x reference)
import functools
import jax
import jax.numpy as jnp
from jax import lax
from jax.experimental import pallas as pl
from jax.experimental.pallas import tpu as pltpu
from jax.experimental.pallas import tpu_sc as plsc

EPS = 1e-6
B = 16384
C = 24
NW = 32
TB = B // NW
L = 16
CHUNKS = TB // L
NPAR = 104

_mesh = plsc.VectorSubcoreMesh(core_axis_name="c", subcore_axis_name="s")


@functools.partial(
    pl.kernel,
    mesh=_mesh,
    out_type=jax.ShapeDtypeStruct((NW, C, TB), jnp.float32),
    scratch_types=[
        pltpu.VMEM((C, TB), jnp.float32),
        pltpu.VMEM((C, TB), jnp.float32),
        pltpu.VMEM((NPAR, L), jnp.float32),
        pltpu.SemaphoreType.DMA,
        pltpu.SemaphoreType.DMA,
    ],
)
def _bayes_fwd(ev_hbm, par_hbm, out_hbm, ev_v, out_v, par_v, sem_in, sem_out):
    wid = lax.axis_index("s") * 2 + lax.axis_index("c")

    H = TB // 2
    in0 = pltpu.async_copy(
        ev_hbm.at[wid, :, pl.ds(0, H)], ev_v.at[:, pl.ds(0, H)], sem_in)
    in1 = pltpu.async_copy(
        ev_hbm.at[wid, :, pl.ds(H, H)], ev_v.at[:, pl.ds(H, H)], sem_in)
    pltpu.sync_copy(par_hbm, par_v)

    def apply_ev(ev, m):
        vals = jnp.minimum(jnp.maximum(ev, 0.0), 1.0) + EPS
        return jnp.where(ev >= 0.0, vals, m)

    def clip01(x):
        return jnp.minimum(jnp.maximum(x, EPS), 1.0 - EPS)

    def chunk_body(row):
        sl = pl.ds(row, L)

        p = [apply_ev(ev_v[j, sl], par_v[j]) for j in range(8)]
        a = [clip01(x) for x in p]
        na = [1.0 - x for x in a]

        q = []
        for n in range(8):
            s0, s1, s2, s3 = (par_v[8 + 4 * n + t] for t in range(4))
            x, y = a[n], a[(n + 1) % 8]
            nx, ny = na[n], na[(n + 1) % 8]
            m = nx * (s0 * ny + s1 * y) + x * (s2 * ny + s3 * y)
            q.append(apply_ev(ev_v[8 + n, sl], m))
        b = [clip01(x) for x in q]
        nb = [1.0 - x for x in b]

        r = []
        for n in range(8):
            s = [par_v[40 + 8 * n + t] for t in range(8)]
            x, y, z = b[n], b[(n + 1) % 8], b[(n + 2) % 8]
            nx, ny, nz = nb[n], nb[(n + 1) % 8], nb[(n + 2) % 8]
            m = (nx * (ny * (s[0] * nz + s[1] * z) + y * (s[2] * nz + s[3] * z))
                 + x * (ny * (s[4] * nz + s[5] * z) + y * (s[6] * nz + s[7] * z)))
            r.append(apply_ev(ev_v[16 + n, sl], m))

        for j, col in enumerate(p + q + r):
            out_v[j, sl] = col

    in0.wait()
    plsc.parallel_loop(0, H, L, unroll=1)(chunk_body)
    out0 = pltpu.async_copy(
        out_v.at[:, pl.ds(0, H)], out_hbm.at[wid, :, pl.ds(0, H)], sem_out)
    in1.wait()
    plsc.parallel_loop(H, TB, L, unroll=1)(chunk_body)
    out1 = pltpu.async_copy(
        out_v.at[:, pl.ds(H, H)], out_hbm.at[wid, :, pl.ds(H, H)], sem_out)
    out0.wait()
    out1.wait()


def kernel(evidence, logits_roots, logits_l2, logits_l3):
    probs = jax.nn.sigmoid(jnp.concatenate(
        [logits_roots.reshape(-1), logits_l2.reshape(-1), logits_l3.reshape(-1)]
    ).astype(jnp.float32))
    par = jnp.broadcast_to(probs[:, None], (NPAR, L))
    ev_tiled = evidence.reshape(NW, TB, C).transpose(0, 2, 1)
    out = _bayes_fwd(ev_tiled, par)
    return out.transpose(0, 2, 1).reshape(B, C)


# speedup vs baseline: 1.0741x; 1.0741x over previous
import functools
import jax
import jax.numpy as jnp
from jax import lax
from jax.experimental import pallas as pl
from jax.experimental.pallas import tpu as pltpu
from jax.experimental.pallas import tpu_sc as plsc

EPS = 1e-6
B = 16384
C = 24
NW = 32
TB = B // NW
L = 16
CHUNKS = TB // L
NPV = 7

_mesh = plsc.VectorSubcoreMesh(core_axis_name="c", subcore_axis_name="s")


@functools.partial(
    pl.kernel,
    mesh=_mesh,
    out_type=jax.ShapeDtypeStruct((NW, C, TB), jnp.float32),
    scratch_types=[
        pltpu.VMEM((C, TB), jnp.float32),
        pltpu.VMEM((C, TB), jnp.float32),
        pltpu.VMEM((NPV, L), jnp.float32),
        pltpu.SemaphoreType.DMA,
        pltpu.SemaphoreType.DMA,
    ],
)
def _bayes_fwd(ev_hbm, par_hbm, out_hbm, ev_v, out_v, par_v, sem_in, sem_out):
    wid = lax.axis_index("s") * 2 + lax.axis_index("c")

    H = TB // 2
    in0 = pltpu.async_copy(
        ev_hbm.at[wid, :, pl.ds(0, H)], ev_v.at[:, pl.ds(0, H)], sem_in)
    in1 = pltpu.async_copy(
        ev_hbm.at[wid, :, pl.ds(H, H)], ev_v.at[:, pl.ds(H, H)], sem_in)
    pltpu.sync_copy(par_hbm, par_v)

    def apply_ev(ev, m):
        return jnp.where(ev >= 0.0, jnp.minimum(ev, 1.0) + EPS, m)

    def clip01(x):
        return jnp.minimum(jnp.maximum(x, EPS), 1.0 - EPS)

    def chunk_body(i, carry):
        sl = pl.ds(i * L, L)
        pk = [par_v[t] for t in range(NPV)]

        def P(k):
            idx = jnp.broadcast_to(jnp.int32(k % L), (L,))
            return jnp.take_along_axis(pk[k // L], idx, axis=0)

        p = [apply_ev(ev_v[j, sl], P(j)) for j in range(8)]
        a = [clip01(x) for x in p]

        q = []
        for n in range(8):
            c0, c1, c2, c3 = (P(8 + 4 * n + t) for t in range(4))
            x, y = a[n], a[(n + 1) % 8]
            m = c0 + c1 * x + c2 * y + c3 * (x * y)
            q.append(apply_ev(ev_v[8 + n, sl], m))
        b = [clip01(x) for x in q]

        r = []
        for n in range(8):
            d = [P(40 + 8 * n + t) for t in range(8)]
            x, y, z = b[n], b[(n + 1) % 8], b[(n + 2) % 8]
            xy = x * y
            u = d[0] + d[1] * x + d[2] * y + d[4] * xy
            v = d[3] + d[5] * x + d[6] * y + d[7] * xy
            r.append(apply_ev(ev_v[16 + n, sl], u + v * z))

        for j, col in enumerate(p + q + r):
            out_v[j, sl] = col
        return carry

    in0.wait()
    lax.fori_loop(0, CHUNKS // 2, chunk_body, 0)
    out0 = pltpu.async_copy(
        out_v.at[:, pl.ds(0, H)], out_hbm.at[wid, :, pl.ds(0, H)], sem_out)
    in1.wait()
    lax.fori_loop(CHUNKS // 2, CHUNKS, chunk_body, 0)
    out1 = pltpu.async_copy(
        out_v.at[:, pl.ds(H, H)], out_hbm.at[wid, :, pl.ds(H, H)], sem_out)
    out0.wait()
    out1.wait()


def _coefficients(logits_roots, logits_l2, logits_l3):
    pr = jax.nn.sigmoid(logits_roots.astype(jnp.float32))[:, 0]
    s2 = jax.nn.sigmoid(logits_l2.astype(jnp.float32))
    s3 = jax.nn.sigmoid(logits_l3.astype(jnp.float32))
    c = jnp.stack([
        s2[:, 0],
        s2[:, 2] - s2[:, 0],
        s2[:, 1] - s2[:, 0],
        s2[:, 3] - s2[:, 2] - s2[:, 1] + s2[:, 0],
    ], axis=1)
    d = jnp.stack([
        s3[:, 0],
        s3[:, 4] - s3[:, 0],
        s3[:, 2] - s3[:, 0],
        s3[:, 1] - s3[:, 0],
        s3[:, 6] - s3[:, 4] - s3[:, 2] + s3[:, 0],
        s3[:, 5] - s3[:, 4] - s3[:, 1] + s3[:, 0],
        s3[:, 3] - s3[:, 2] - s3[:, 1] + s3[:, 0],
        s3[:, 7] - s3[:, 6] - s3[:, 5] - s3[:, 3]
        + s3[:, 4] + s3[:, 2] + s3[:, 1] - s3[:, 0],
    ], axis=1)
    flat = jnp.concatenate([pr, c.reshape(-1), d.reshape(-1)])
    return jnp.pad(flat, (0, NPV * L - flat.shape[0])).reshape(NPV, L)


def kernel(evidence, logits_roots, logits_l2, logits_l3):
    par = _coefficients(logits_roots, logits_l2, logits_l3)
    ev_tiled = evidence.reshape(NW, TB, C).transpose(0, 2, 1)
    out = _bayes_fwd(ev_tiled, par)
    return out.transpose(0, 2, 1).reshape(B, C)
